# Initial kernel scaffold; baseline (speedup 1.0000x reference)
import functools

import jax
import jax.numpy as jnp
from jax import lax
from jax.experimental import pallas as pl
from jax.experimental.pallas import tpu as pltpu
from jax.experimental.pallas import tpu_sc as plsc

EMBED_DIM = 64
NUM_CORES = 2
NUM_SUBCORES = 16
NUM_WORKERS = NUM_CORES * NUM_SUBCORES
CHUNK = 512


@functools.lru_cache(maxsize=None)
def _build_gather(batch, embed_dim):
    assert batch % (NUM_WORKERS * CHUNK) == 0
    per_worker = batch // NUM_WORKERS
    n_chunks = per_worker // CHUNK
    assert n_chunks >= 2 and n_chunks % 2 == 0

    mesh = plsc.VectorSubcoreMesh(
        core_axis_name="c",
        subcore_axis_name="s",
        num_cores=NUM_CORES,
        num_subcores=NUM_SUBCORES,
    )

    @functools.partial(
        pl.kernel,
        out_type=jax.ShapeDtypeStruct((batch, embed_dim), jnp.float32),
        mesh=mesh,
        scratch_types=[
            pltpu.VMEM((n_chunks, CHUNK), jnp.int32),
            pltpu.VMEM((CHUNK, embed_dim), jnp.float32),
            pltpu.VMEM((CHUNK, embed_dim), jnp.float32),
            pltpu.SemaphoreType.DMA,
            pltpu.SemaphoreType.DMA,
        ],
    )
    def gather_kernel(table_hbm, idx_hbm, out_hbm, idx_v, rows0, rows1, g0, g1):
        wid = lax.axis_index("s") * NUM_CORES + lax.axis_index("c")
        base = wid * per_worker
        pltpu.sync_copy(idx_hbm.at[pl.ds(wid * n_chunks, n_chunks)], idx_v)

        rows = (rows0, rows1)
        gsem = (g0, g1)

        def start_gather(c, b):
            pltpu.async_copy(table_hbm.at[idx_v.at[c]], rows[b], gsem[b])

        def wait_gather(c, b):
            pltpu.make_async_copy(table_hbm.at[idx_v.at[c]], rows[b], gsem[b]).wait()

        def process(c, b, issue_next):
            wait_gather(c, b)
            pltpu.sync_copy(rows[b], out_hbm.at[pl.ds(base + c * CHUNK, CHUNK)])
            if issue_next:
                start_gather(c + 2, b)

        start_gather(0, 0)
        start_gather(1, 1)

        @pl.loop(0, n_chunks - 2, step=2)
        def _(j):
            process(j, 0, True)
            process(j + 1, 1, True)

        process(n_chunks - 2, 0, False)
        process(n_chunks - 1, 1, False)

    return gather_kernel


def kernel(vis_feat, captions, lengths, table):
    batch, seq = captions.shape
    total = batch * seq
    idx2d = captions.reshape(total // CHUNK, CHUNK)
    gather = _build_gather(total, table.shape[1])
    out = gather(table, idx2d)
    return out.reshape(batch, seq, table.shape[1])


# speedup vs baseline: 1.8733x; 1.8733x over previous
import functools

import jax
import jax.numpy as jnp
from jax import lax
from jax.experimental import pallas as pl
from jax.experimental.pallas import tpu as pltpu
from jax.experimental.pallas import tpu_sc as plsc

EMBED_DIM = 64
NUM_CORES = 2
NUM_SUBCORES = 16
NUM_WORKERS = NUM_CORES * NUM_SUBCORES
CHUNK = 512


@functools.lru_cache(maxsize=None)
def _build_gather(batch, embed_dim):
    assert batch % (NUM_WORKERS * CHUNK) == 0
    per_worker = batch // NUM_WORKERS
    n_chunks = per_worker // CHUNK
    assert n_chunks >= 2 and n_chunks % 2 == 0

    mesh = plsc.VectorSubcoreMesh(
        core_axis_name="c",
        subcore_axis_name="s",
        num_cores=NUM_CORES,
        num_subcores=NUM_SUBCORES,
    )

    @functools.partial(
        pl.kernel,
        out_type=jax.ShapeDtypeStruct((batch, embed_dim), jnp.float32),
        mesh=mesh,
        compiler_params=pltpu.CompilerParams(use_tc_tiling_on_sc=False),
        scratch_types=[
            pltpu.VMEM((per_worker,), jnp.int32),
            pltpu.VMEM((CHUNK, embed_dim), jnp.float32),
            pltpu.VMEM((CHUNK, embed_dim), jnp.float32),
            pltpu.SemaphoreType.DMA,
            pltpu.SemaphoreType.DMA,
        ],
    )
    def gather_kernel(table_hbm, idx_hbm, out_hbm, idx_v, rows0, rows1, g0, g1):
        wid = lax.axis_index("s") * NUM_CORES + lax.axis_index("c")
        base = wid * per_worker
        pltpu.sync_copy(idx_hbm.at[pl.ds(base, per_worker)], idx_v)

        rows = (rows0, rows1)
        gsem = (g0, g1)

        def start_gather(c, b):
            pltpu.async_copy(
                table_hbm.at[idx_v.at[pl.ds(c * CHUNK, CHUNK)]], rows[b], gsem[b]
            )

        def wait_gather(c, b):
            pltpu.make_async_copy(
                table_hbm.at[idx_v.at[pl.ds(c * CHUNK, CHUNK)]], rows[b], gsem[b]
            ).wait()

        def process(c, b, issue_next):
            wait_gather(c, b)
            pltpu.sync_copy(rows[b], out_hbm.at[pl.ds(base + c * CHUNK, CHUNK)])
            if issue_next:
                start_gather(c + 2, b)

        start_gather(0, 0)
        start_gather(1, 1)

        @pl.loop(0, n_chunks - 2, step=2)
        def _(j):
            process(j, 0, True)
            process(j + 1, 1, True)

        process(n_chunks - 2, 0, False)
        process(n_chunks - 1, 1, False)

    return gather_kernel


def kernel(vis_feat, captions, lengths, table):
    batch, seq = captions.shape
    total = batch * seq
    idx_flat = captions.reshape(total)
    gather = _build_gather(total, table.shape[1])
    out = gather(table, idx_flat)
    return out.reshape(batch, seq, table.shape[1])


# speedup vs baseline: 1.8753x; 1.0011x over previous
import functools

import jax
import jax.numpy as jnp
from jax import lax
from jax.experimental import pallas as pl
from jax.experimental.pallas import tpu as pltpu
from jax.experimental.pallas import tpu_sc as plsc

EMBED_DIM = 64
NUM_CORES = 2
NUM_SUBCORES = 16
NUM_WORKERS = NUM_CORES * NUM_SUBCORES
CHUNK = 320
NBUF = 4


@functools.lru_cache(maxsize=None)
def _build_gather(batch, embed_dim):
    assert batch % (NUM_WORKERS * CHUNK) == 0
    per_worker = batch // NUM_WORKERS
    n_chunks = per_worker // CHUNK
    assert n_chunks >= NBUF + 1
    assert (n_chunks - NBUF) % NBUF == 0

    mesh = plsc.VectorSubcoreMesh(
        core_axis_name="c",
        subcore_axis_name="s",
        num_cores=NUM_CORES,
        num_subcores=NUM_SUBCORES,
    )

    @functools.partial(
        pl.kernel,
        out_type=jax.ShapeDtypeStruct((batch, embed_dim), jnp.float32),
        mesh=mesh,
        compiler_params=pltpu.CompilerParams(use_tc_tiling_on_sc=False),
        scratch_types=[
            pltpu.VMEM((per_worker,), jnp.int32),
            *[pltpu.VMEM((CHUNK, embed_dim), jnp.float32) for _ in range(NBUF)],
            *[pltpu.SemaphoreType.DMA for _ in range(2 * NBUF)],
        ],
    )
    def gather_kernel(table_hbm, idx_hbm, out_hbm, idx_v, *bufs_and_sems):
        rows = bufs_and_sems[:NBUF]
        gsem = bufs_and_sems[NBUF : 2 * NBUF]
        osem = bufs_and_sems[2 * NBUF :]

        wid = lax.axis_index("s") * NUM_CORES + lax.axis_index("c")
        base = wid * per_worker
        pltpu.sync_copy(idx_hbm.at[pl.ds(base, per_worker)], idx_v)

        def start_gather(c, b):
            pltpu.async_copy(
                table_hbm.at[idx_v.at[pl.ds(c * CHUNK, CHUNK)]], rows[b], gsem[b]
            )

        def wait_gather(c, b):
            pltpu.make_async_copy(
                table_hbm.at[idx_v.at[pl.ds(c * CHUNK, CHUNK)]], rows[b], gsem[b]
            ).wait()

        def start_out(c, b):
            pltpu.async_copy(
                rows[b], out_hbm.at[pl.ds(base + c * CHUNK, CHUNK)], osem[b]
            )

        def wait_out(c, b):
            pltpu.make_async_copy(
                rows[b], out_hbm.at[pl.ds(base + c * CHUNK, CHUNK)], osem[b]
            ).wait()

        for c in range(NBUF - 1):
            start_gather(c, c)

        wait_gather(0, 0)
        start_out(0, 0)
        start_gather(NBUF - 1, NBUF - 1)

        @pl.loop(1, n_chunks - NBUF + 1, step=NBUF)
        def _(j):
            for i in range(NBUF):
                c = j + i
                b = (1 + i) % NBUF
                b_next = i % NBUF
                wait_gather(c, b)
                start_out(c, b)
                wait_out(c - 1, b_next)
                start_gather(c + NBUF - 1, b_next)

        for k in range(NBUF - 1, 0, -1):
            c = n_chunks - k
            wait_gather(c, c % NBUF)
            start_out(c, c % NBUF)

        for c in range(n_chunks - NBUF, n_chunks):
            wait_out(c, c % NBUF)

    return gather_kernel


def kernel(vis_feat, captions, lengths, table):
    batch, seq = captions.shape
    total = batch * seq
    idx_flat = captions.reshape(total)
    gather = _build_gather(total, table.shape[1])
    out = gather(table, idx_flat)
    return out.reshape(batch, seq, table.shape[1])
